# baseline (speedup 1.0000x reference)
import jax
import jax.numpy as jnp
from jax.experimental import pallas as pl
from jax.experimental.pallas import tpu as pltpu

_FEAT = 64
_EMB = 128
_BLOCK2 = 10000


def _fused_kernel(x_ref, w1_ref, b1_ref, w2_ref, b2_ref, o_ref):
    x = x_ref[...]
    h = jnp.dot(x, w1_ref[...], preferred_element_type=jnp.float32) + b1_ref[...]
    h = jnp.maximum(h, 0.0)
    e = jnp.dot(h, w2_ref[...], preferred_element_type=jnp.float32) + b2_ref[...]
    ea = e[:, :_EMB]
    eb = e[:, _EMB:]
    sqa = jnp.sum(ea * ea, axis=-1, keepdims=True)
    sqb = jnp.sum(eb * eb, axis=-1, keepdims=True)
    oa = jnp.where(sqa > 0.0, ea * jax.lax.rsqrt(sqa), 0.0)
    ob = jnp.where(sqb > 0.0, eb * jax.lax.rsqrt(sqb), 0.0)
    o_ref[...] = jnp.concatenate([oa, ob], axis=-1)


def kernel(x, W1, b1, W2, b2):
    n_rows = x.shape[0]
    n2 = n_rows // 2
    x2 = x.reshape(n2, 2 * _FEAT)

    zero1 = jnp.zeros((_FEAT, _FEAT), dtype=jnp.float32)
    w1d = jnp.block([[W1, zero1], [zero1, W1]])
    zero2 = jnp.zeros((_FEAT, _EMB), dtype=jnp.float32)
    w2d = jnp.block([[W2, zero2], [zero2, W2]])
    b1d = jnp.concatenate([b1, b1]).reshape(1, 2 * _FEAT)
    b2d = jnp.concatenate([b2, b2]).reshape(1, 2 * _EMB)

    grid = (n2 // _BLOCK2,)
    out2 = pl.pallas_call(
        _fused_kernel,
        grid=grid,
        in_specs=[
            pl.BlockSpec((_BLOCK2, 2 * _FEAT), lambda i: (i, 0)),
            pl.BlockSpec((2 * _FEAT, 2 * _FEAT), lambda i: (0, 0)),
            pl.BlockSpec((1, 2 * _FEAT), lambda i: (0, 0)),
            pl.BlockSpec((2 * _FEAT, 2 * _EMB), lambda i: (0, 0)),
            pl.BlockSpec((1, 2 * _EMB), lambda i: (0, 0)),
        ],
        out_specs=pl.BlockSpec((_BLOCK2, 2 * _EMB), lambda i: (i, 0)),
        out_shape=jax.ShapeDtypeStruct((n2, 2 * _EMB), jnp.float32),
        compiler_params=pltpu.CompilerParams(
            dimension_semantics=("arbitrary",),
            vmem_limit_bytes=56 * 1024 * 1024,
        ),
    )(x2, w1d, b1d, w2d, b2d)
    return out2.reshape(n_rows, _EMB)


# speedup vs baseline: 2.0427x; 2.0427x over previous
import jax
import jax.numpy as jnp
from jax.experimental import pallas as pl
from jax.experimental.pallas import tpu as pltpu

_FEAT = 64
_EMB = 128
_CH = 8000
_NC = 125


def _fused_kernel(w1_ref, b1_ref, w2_ref, b2_ref, x_hbm, o_hbm,
                  xb, ob, isem, osem):
    k = pl.program_id(0)
    slot = jax.lax.rem(k, 2)

    @pl.when(k == 0)
    def _():
        pltpu.make_async_copy(
            x_hbm.at[pl.ds(0, _CH)], xb.at[0], isem.at[0]).start()
        pltpu.make_async_copy(
            x_hbm.at[pl.ds(_CH, _CH)], xb.at[1], isem.at[1]).start()

    pltpu.make_async_copy(
        x_hbm.at[pl.ds(k * _CH, _CH)], xb.at[slot], isem.at[slot]).wait()

    @pl.when(k >= 2)
    def _():
        pltpu.make_async_copy(
            ob.at[slot], o_hbm.at[pl.ds((k - 2) * _CH, _CH)],
            osem.at[slot]).wait()

    x = xb[slot]
    h = jnp.dot(x, w1_ref[...], preferred_element_type=jnp.float32) + b1_ref[...]
    h = jnp.maximum(h, 0.0)
    e = jnp.dot(h, w2_ref[...], preferred_element_type=jnp.float32) + b2_ref[...]
    sq = jnp.sum(e * e, axis=-1, keepdims=True)
    ob[slot] = jnp.where(sq > 0.0, e * jax.lax.rsqrt(sq), 0.0)

    pltpu.make_async_copy(
        ob.at[slot], o_hbm.at[pl.ds(k * _CH, _CH)], osem.at[slot]).start()

    @pl.when(k + 2 < _NC)
    def _():
        pltpu.make_async_copy(
            x_hbm.at[pl.ds((k + 2) * _CH, _CH)], xb.at[slot],
            isem.at[slot]).start()

    @pl.when(k == _NC - 1)
    def _():
        nslot = 1 - slot
        pltpu.make_async_copy(
            ob.at[nslot], o_hbm.at[pl.ds((_NC - 2) * _CH, _CH)],
            osem.at[nslot]).wait()
        pltpu.make_async_copy(
            ob.at[slot], o_hbm.at[pl.ds((_NC - 1) * _CH, _CH)],
            osem.at[slot]).wait()


def kernel(x, W1, b1, W2, b2):
    n_rows = x.shape[0]
    return pl.pallas_call(
        _fused_kernel,
        grid=(_NC,),
        in_specs=[
            pl.BlockSpec((_FEAT, _FEAT), lambda i: (0, 0)),
            pl.BlockSpec((1, _FEAT), lambda i: (0, 0)),
            pl.BlockSpec((_FEAT, _EMB), lambda i: (0, 0)),
            pl.BlockSpec((1, _EMB), lambda i: (0, 0)),
            pl.BlockSpec(memory_space=pl.ANY),
            ],
        out_specs=pl.BlockSpec(memory_space=pl.ANY),
        out_shape=jax.ShapeDtypeStruct((n_rows, _EMB), jnp.float32),
        scratch_shapes=[
            pltpu.VMEM((2, _CH, _FEAT), jnp.float32),
            pltpu.VMEM((2, _CH, _EMB), jnp.float32),
            pltpu.SemaphoreType.DMA((2,)),
            pltpu.SemaphoreType.DMA((2,)),
        ],
        compiler_params=pltpu.CompilerParams(
            dimension_semantics=("arbitrary",),
            vmem_limit_bytes=56 * 1024 * 1024,
        ),
    )(W1, b1.reshape(1, _FEAT), W2, b2.reshape(1, _EMB), x)
